# baseline (speedup 1.0000x reference)
import jax
import jax.numpy as jnp
from jax import lax
from jax.experimental import pallas as pl
from jax.experimental.pallas import tpu as pltpu
from jax.experimental.pallas import tpu_sc as plsc

D_MODEL = 128
BATCH = 16384
LANES = 16

_info = plsc.get_sparse_core_info()
NUM_CORES = _info.num_cores
NUM_SUBCORES = _info.num_subcores
NW = NUM_CORES * NUM_SUBCORES
BPW = BATCH // NW

CHS = ((0, 64), (64, 64), (128, 128), (256, 128), (384, 128))
NCH = len(CHS)
BUFROWS = 128
DEPTH = 3


def _body(x_hbm, t_hbm, pe_hbm, out_hbm, idx_v, x_big, *rest):
    pe_bufs = rest[:DEPTH]
    gsems = rest[DEPTH:2 * DEPTH]
    xsems = rest[2 * DEPTH:2 * DEPTH + NCH]
    osem = rest[2 * DEPTH + NCH]

    wid = lax.axis_index("s") * NUM_CORES + lax.axis_index("c")
    base = wid * BPW
    pltpu.sync_copy(t_hbm.at[pl.ds(base, BPW)], idx_v)

    def fire_gather(ci, buf, sem):
        st, n = CHS[ci]
        return pltpu.async_copy(
            pe_hbm.at[idx_v.at[pl.ds(st, n)]], buf.at[pl.ds(0, n)], sem)

    copies = [None] * NCH
    xcopies = [None] * NCH
    for ci in range(NCH):
        st, n = CHS[ci]
        if ci < DEPTH:
            copies[ci] = fire_gather(ci, pe_bufs[ci], gsems[ci])
        xcopies[ci] = pltpu.async_copy(
            x_hbm.at[pl.ds(base + st, n)],
            x_big.at[pl.ds(st, n)], xsems[ci])

    stores = []
    for ci in range(NCH):
        st, n = CHS[ci]
        k = ci % DEPTH
        with jax.named_scope(f"wait{ci}"):
            copies[ci].wait()
            xcopies[ci].wait()
        pe_b = pe_bufs[k]

        with jax.named_scope(f"add{ci}"):
            @plsc.parallel_loop(0, n, unroll=2)
            def _row(r):
                xr = st + r
                for j in range(D_MODEL // LANES):
                    sl = pl.ds(j * LANES, LANES)
                    plsc.addupdate(x_big.at[xr, sl], pe_b[r, sl])

        if ci + DEPTH < NCH:
            copies[ci + DEPTH] = fire_gather(ci + DEPTH, pe_b, gsems[k])
        stores.append(pltpu.async_copy(
            x_big.at[pl.ds(st, n)],
            out_hbm.at[pl.ds(base + st, n)], osem))
    with jax.named_scope("drain"):
        for s in stores:
            s.wait()


@jax.jit
def _run(x, t, pe2d):
    mesh = plsc.VectorSubcoreMesh(core_axis_name="c", subcore_axis_name="s")
    k = pl.kernel(
        _body,
        mesh=mesh,
        out_type=jax.ShapeDtypeStruct((BATCH, D_MODEL), jnp.float32),
        scratch_types=(
            [pltpu.VMEM((BPW,), jnp.int32),
             pltpu.VMEM((BPW, D_MODEL), jnp.float32)]
            + [pltpu.VMEM((BUFROWS, D_MODEL), jnp.float32)] * DEPTH
            + [pltpu.SemaphoreType.DMA] * (DEPTH + NCH + 1)
        ),
    )
    return k(x, t, pe2d)


def kernel(x, t, pe):
    out = _run(x, t.astype(jnp.int32), pe.reshape(pe.shape[1], pe.shape[2]))
    return out[None]
